# baseline (speedup 1.0000x reference)
import functools

import jax
import jax.numpy as jnp
from jax import lax
from jax.experimental import pallas as pl
from jax.experimental.pallas import tpu as pltpu
from jax.experimental.pallas import tpu_sc as plsc

_BATCH = 16384
_SEQ = 200
_D = 32
_NC = 2
_NS = 16
_NW = _NC * _NS
_ROWS_W = _BATCH // _NW
_DEPTH = 4


def _emb_body(table_hbm, x_hbm, out_hbm, idx_v, rows_v, si, sg, so):
    wid = lax.axis_index("s") * _NC + lax.axis_index("c")
    row0 = wid * _ROWS_W

    def start_idx(r, d):
        pltpu.async_copy(x_hbm.at[row0 + r], idx_v[d], si[d])

    def wait_idx(d):
        pltpu.make_async_copy(x_hbm.at[0], idx_v[d], si[d]).wait()

    def start_gather(d):
        pltpu.async_copy(table_hbm.at[idx_v[d]], rows_v[d], sg[d])

    def wait_gather(d):
        pltpu.make_async_copy(table_hbm.at[idx_v[d]], rows_v[d], sg[d]).wait()

    def start_out(r, d):
        pltpu.async_copy(rows_v[d], out_hbm.at[row0 + r], so[d])

    def wait_out(d):
        pltpu.make_async_copy(rows_v[d], out_hbm.at[0], so[d]).wait()

    for d in range(_DEPTH):
        start_idx(d, d)
    wait_idx(0)
    start_gather(0)
    for r in (1, 2, 3):
        d = r & 3
        wait_idx(d)
        start_gather(d)
        wait_gather(d - 1 & 3)
        start_out(r - 1, d - 1 & 3)
        start_idx(r + 3, d - 1 & 3)

    def step_fn(g, carry):
        for b in range(_DEPTH):
            r = _DEPTH * g + b
            d1 = (b - 1) & 3
            wait_idx(b)
            wait_out(b)
            start_gather(b)
            wait_gather(d1)
            start_out(r - 1, d1)

            @pl.when(r + 3 < _ROWS_W)
            def _():
                start_idx(r + 3, d1)
        return carry

    lax.fori_loop(1, _ROWS_W // _DEPTH, step_fn, 0)

    wait_gather(3)
    start_out(_ROWS_W - 1, 3)
    for d in range(_DEPTH):
        wait_out(d)


@jax.jit
def kernel(x, emb_tim):
    mesh = plsc.VectorSubcoreMesh(core_axis_name="c", subcore_axis_name="s")
    run = functools.partial(
        pl.kernel,
        mesh=mesh,
        out_type=jax.ShapeDtypeStruct((_BATCH, _SEQ, _D), jnp.float32),
        scratch_types=[
            [pltpu.VMEM((_SEQ,), jnp.int32) for _ in range(_DEPTH)],
            [pltpu.VMEM((_SEQ, _D), jnp.float32) for _ in range(_DEPTH)],
            [pltpu.SemaphoreType.DMA for _ in range(_DEPTH)],
            [pltpu.SemaphoreType.DMA for _ in range(_DEPTH)],
            [pltpu.SemaphoreType.DMA for _ in range(_DEPTH)],
        ],
        compiler_params=pltpu.CompilerParams(use_tc_tiling_on_sc=False),
    )(_emb_body)
    out = run(emb_tim, x.astype(jnp.int32))
    return out
